# baseline (speedup 1.0000x reference)
import jax
import jax.numpy as jnp
from jax.experimental import pallas as pl
from jax.experimental.pallas import tpu as pltpu

_BQ = 4096
_CK = 128


def _body(q_ref, k_ref, qsq_ref, ksq_ref, idx_ref, dist_ref):
    q = q_ref[...]
    k = k_ref[...]
    q_sq = qsq_ref[...]
    k_sq = ksq_ref[...]
    n_keys = k.shape[0]
    runm = runi = None
    for c in range(0, n_keys, _CK):
        dots_c = jax.lax.dot_general(
            k[c:c + _CK], q, (((1,), (1,)), ((), ())),
            preferred_element_type=jnp.float32,
        )
        sc = (q_sq + k_sq[c:c + _CK]) - 2.0 * dots_c
        mc = jnp.min(sc, axis=0)
        iota = jax.lax.broadcasted_iota(jnp.int32, sc.shape, 0)
        ic = jnp.min(jnp.where(sc == mc[None, :], iota, _CK), axis=0) + c
        if runm is None:
            runm, runi = mc, ic
        else:
            better = mc < runm
            runi = jnp.where(better, ic, runi)
            runm = jnp.where(better, mc, runm)
    idx_ref[0, 0, :] = runi
    dist_ref[0, 0, :] = jnp.sqrt(jnp.maximum(runm, 1e-12))


def kernel(queries, keys):
    Q, D = queries.shape
    K, _ = keys.shape
    q_sq = jnp.sum(queries * queries, axis=1)[None, :]
    k_sq = jnp.sum(keys * keys, axis=1)[:, None]
    grid = Q // _BQ
    idx, dist = pl.pallas_call(
        _body,
        grid=(grid,),
        in_specs=[
            pl.BlockSpec((_BQ, D), lambda i: (i, 0)),
            pl.BlockSpec((K, D), lambda i: (0, 0)),
            pl.BlockSpec((1, _BQ), lambda i: (0, i)),
            pl.BlockSpec((K, 1), lambda i: (0, 0)),
        ],
        out_specs=[
            pl.BlockSpec((1, 1, _BQ), lambda i: (i, 0, 0)),
            pl.BlockSpec((1, 1, _BQ), lambda i: (i, 0, 0)),
        ],
        out_shape=[
            jax.ShapeDtypeStruct((grid, 1, _BQ), jnp.int32),
            jax.ShapeDtypeStruct((grid, 1, _BQ), jnp.float32),
        ],
        compiler_params=pltpu.CompilerParams(
            dimension_semantics=("parallel",),
        ),
    )(queries, keys, q_sq, k_sq)
    return idx.reshape(Q), dist.reshape(Q)


# speedup vs baseline: 1.0023x; 1.0023x over previous
import jax
import jax.numpy as jnp
from jax.experimental import pallas as pl

_BQ = 4096
_CK = 128


def _body(q_ref, k_ref, qsq_ref, ksq_ref, idx_ref, dist_ref):
    q = q_ref[...]
    k = k_ref[...]
    q_sq = qsq_ref[...]
    k_sq = ksq_ref[...]
    n_keys = k.shape[0]
    runm = runi = None
    for c in range(0, n_keys, _CK):
        dots_c = jax.lax.dot_general(
            k[c:c + _CK], q, (((1,), (1,)), ((), ())),
            preferred_element_type=jnp.float32,
        )
        sc = (q_sq + k_sq[c:c + _CK]) - 2.0 * dots_c
        mc = jnp.min(sc, axis=0)
        iota = jax.lax.broadcasted_iota(jnp.int32, sc.shape, 0)
        ic = jnp.min(jnp.where(sc == mc[None, :], iota, _CK), axis=0) + c
        if runm is None:
            runm, runi = mc, ic
        else:
            better = mc < runm
            runi = jnp.where(better, ic, runi)
            runm = jnp.where(better, mc, runm)
    idx_ref[0, 0, :] = runi
    dist_ref[0, 0, :] = jnp.sqrt(jnp.maximum(runm, 1e-12))


def kernel(queries, keys):
    Q, D = queries.shape
    K, _ = keys.shape
    q_sq = jnp.sum(queries * queries, axis=1)[None, :]
    k_sq = jnp.sum(keys * keys, axis=1)[:, None]
    grid = Q // _BQ
    idx, dist = pl.pallas_call(
        _body,
        grid=(grid,),
        in_specs=[
            pl.BlockSpec((_BQ, D), lambda i: (i, 0)),
            pl.BlockSpec((K, D), lambda i: (0, 0)),
            pl.BlockSpec((1, _BQ), lambda i: (0, i)),
            pl.BlockSpec((K, 1), lambda i: (0, 0)),
        ],
        out_specs=[
            pl.BlockSpec((1, 1, _BQ), lambda i: (i, 0, 0)),
            pl.BlockSpec((1, 1, _BQ), lambda i: (i, 0, 0)),
        ],
        out_shape=[
            jax.ShapeDtypeStruct((grid, 1, _BQ), jnp.int32),
            jax.ShapeDtypeStruct((grid, 1, _BQ), jnp.float32),
        ],
    )(queries, keys, q_sq, k_sq)
    return idx.reshape(Q), dist.reshape(Q)
